# Initial kernel scaffold; baseline (speedup 1.0000x reference)
import jax
import jax.numpy as jnp
from jax.experimental import pallas as pl


def kernel(batch, emb_table):
    raise NotImplementedError("write your pallas kernel here")


# speedup vs baseline: 5.8016x; 5.8016x over previous
import functools

import jax
import jax.numpy as jnp
from jax import lax
from jax.experimental import pallas as pl
from jax.experimental.pallas import tpu as pltpu
from jax.experimental.pallas import tpu_sc as plsc

D = 128
B_ROWS = 1024
B_COLS = 200
B = B_ROWS * B_COLS
NC = 2
NS = 16
NW = NC * NS
CH = 128
B_PER_W = B // NW
N_CHUNKS = B_PER_W // CH

_mesh = plsc.VectorSubcoreMesh(core_axis_name="c", subcore_axis_name="s")


@functools.partial(
    pl.kernel,
    mesh=_mesh,
    out_type=jax.ShapeDtypeStruct((B, D), jnp.float32),
    scratch_types=[
        pltpu.VMEM((N_CHUNKS, CH), jnp.int32),
        pltpu.VMEM((CH, D), jnp.float32),
        pltpu.SemaphoreType.DMA,
    ],
)
def _gather_kernel(idx_hbm, table_hbm, out_hbm, idx_v, rows_v, sem):
    wid = lax.axis_index("s") * NC + lax.axis_index("c")
    pltpu.sync_copy(idx_hbm.at[wid], idx_v)

    def body(j, carry):
        pltpu.async_copy(table_hbm.at[idx_v.at[j]], rows_v, sem).wait()
        pltpu.sync_copy(rows_v, out_hbm.at[pl.ds(wid * B_PER_W + j * CH, CH)])
        return carry

    lax.fori_loop(0, N_CHUNKS, body, 0)


def kernel(batch, emb_table):
    idx3 = batch.reshape(NW, N_CHUNKS, CH)
    out = _gather_kernel(idx3, emb_table)
    return out.reshape(B_ROWS, B_COLS, D)


# speedup vs baseline: 8.0135x; 1.3813x over previous
import functools

import jax
import jax.numpy as jnp
from jax import lax
from jax.experimental import pallas as pl
from jax.experimental.pallas import tpu as pltpu
from jax.experimental.pallas import tpu_sc as plsc

D = 128
B_ROWS = 1024
B_COLS = 200
B = B_ROWS * B_COLS
NC = 2
NS = 16
NW = NC * NS
CH = 128
B_PER_W = B // NW
N_CHUNKS = B_PER_W // CH
NB = 5
PD = 3

_mesh = plsc.VectorSubcoreMesh(core_axis_name="c", subcore_axis_name="s")


@functools.partial(
    pl.kernel,
    mesh=_mesh,
    out_type=jax.ShapeDtypeStruct((B, D), jnp.float32),
    scratch_types=[
        pltpu.VMEM((N_CHUNKS, CH), jnp.int32),
        pltpu.VMEM((NB, CH, D), jnp.float32),
        pltpu.SemaphoreType.DMA((NB,)),
        pltpu.SemaphoreType.DMA((NB,)),
    ],
)
def _gather_kernel(idx_hbm, table_hbm, out_hbm, idx_v, rows_v, gsem, wsem):
    wid = lax.axis_index("s") * NC + lax.axis_index("c")
    base = wid * B_PER_W
    pltpu.sync_copy(idx_hbm.at[wid], idx_v)

    for b in range(PD):
        pltpu.async_copy(table_hbm.at[idx_v.at[b]], rows_v.at[b], gsem.at[b])

    def outer(g, carry):
        for b in range(NB):
            j = g * NB + b
            f = j + PD
            fb = (b + PD) % NB
            @pl.when(jnp.logical_and(f >= NB, f < N_CHUNKS))
            def _():
                pltpu.make_async_copy(
                    rows_v.at[fb],
                    out_hbm.at[pl.ds(base + (f - NB) * CH, CH)],
                    wsem.at[fb],
                ).wait()

            @pl.when(f < N_CHUNKS)
            def _():
                pltpu.async_copy(
                    table_hbm.at[idx_v.at[f]], rows_v.at[fb], gsem.at[fb])

            pltpu.make_async_copy(
                table_hbm.at[idx_v.at[j]], rows_v.at[b], gsem.at[b]).wait()
            pltpu.async_copy(
                rows_v.at[b], out_hbm.at[pl.ds(base + j * CH, CH)], wsem.at[b])
        return carry

    lax.fori_loop(0, N_CHUNKS // NB, outer, 0)

    for b in range(NB):
        j = N_CHUNKS - NB + b
        pltpu.make_async_copy(
            rows_v.at[b], out_hbm.at[pl.ds(base + j * CH, CH)], wsem.at[b]
        ).wait()


def kernel(batch, emb_table):
    idx3 = batch.reshape(NW, N_CHUNKS, CH)
    out = _gather_kernel(idx3, emb_table)
    return out.reshape(B_ROWS, B_COLS, D)
